# baseline (speedup 1.0000x reference)
import jax
import jax.numpy as jnp
from jax import lax
from jax.experimental import pallas as pl
from jax.experimental.pallas import tpu as pltpu
from jax.experimental.pallas import tpu_sc as plsc

DIM = 2048
UNIT_DIM = 64
N = 8192
BATCH = 16384
N_CHUNKS = DIM // UNIT_DIM

NUM_CORES = 2
NUM_SUBCORES = 16
NW = NUM_CORES * NUM_SUBCORES
B = BATCH * N_CHUNKS
B_PER_W = B // NW
K = 128
T = B_PER_W // K
NBUF = 8
NROUNDS = T // NBUF
OUT_ROWS = B // 2
OR_PER_W = OUT_ROWS // NW
OR_PER_TILE = K // 2


def _gather_body(table_hbm, gidx_hbm, out_hbm, idx_v, rows_v, *sems):
    gsem = sems[:NBUF]
    ssem = sems[NBUF:]
    wid = lax.axis_index("s") * NUM_CORES + lax.axis_index("c")
    base = wid * B_PER_W
    pltpu.sync_copy(gidx_hbm.at[wid], idx_v)

    for b in range(NBUF):
        pltpu.async_copy(table_hbm.at[idx_v.at[b]], rows_v.at[b], gsem[b])

    def do_slot(r, b, start_next):
        j = r * NBUF + b
        pltpu.make_async_copy(table_hbm.at[idx_v.at[b]], rows_v.at[b],
                              gsem[b]).wait()
        out_slice = out_hbm.at[pl.ds(base + j * K, K)]
        pltpu.async_copy(rows_v.at[b], out_slice, ssem[b])
        pltpu.make_async_copy(rows_v.at[b], out_slice, ssem[b]).wait()
        if start_next:
            pltpu.async_copy(table_hbm.at[idx_v.at[j + NBUF]], rows_v.at[b],
                             gsem[b])

    def round_body(r, carry):
        for b in range(NBUF):
            do_slot(r, b, True)
        return carry

    lax.fori_loop(0, NROUNDS - 1, round_body, 0)
    for b in range(NBUF):
        do_slot(NROUNDS - 1, b, False)


@jax.jit
def _sc_gather(table, gidx):
    mesh = plsc.VectorSubcoreMesh(core_axis_name="c", subcore_axis_name="s")
    return pl.kernel(
        _gather_body,
        out_type=jax.ShapeDtypeStruct((B, UNIT_DIM), jnp.float32),
        mesh=mesh,
        scratch_types=[
            pltpu.VMEM((T, K), jnp.int32),
            pltpu.VMEM((NBUF, K, UNIT_DIM), jnp.float32),
        ] + [pltpu.SemaphoreType.DMA] * (2 * NBUF),
        compiler_params=pltpu.CompilerParams(use_tc_tiling_on_sc=False),
    )(table, gidx)


def kernel(inputs):
    idx_key = jax.random.key(1)
    keys = jax.vmap(lambda i: jax.random.fold_in(idx_key, i))(jnp.arange(N_CHUNKS))
    idx = jax.vmap(lambda k: jax.random.randint(k, (BATCH,), 0, N))(keys)

    gidx = idx.T * N_CHUNKS + jnp.arange(N_CHUNKS, dtype=jnp.int32)[None, :]
    gidx = gidx.reshape(BATCH // 8, 8, N_CHUNKS // 2, 2)
    gidx = gidx.transpose(0, 2, 1, 3).reshape(NW, T, K)

    table = inputs[:N].reshape(N * N_CHUNKS, UNIT_DIM)
    out = _sc_gather(table, gidx)
    out = out.reshape(BATCH // 8, DIM // 128, 8, 128)
    return out.transpose(0, 2, 1, 3).reshape(BATCH, DIM)


# speedup vs baseline: 1.4744x; 1.4744x over previous
import jax
import jax.numpy as jnp
from jax import lax
from jax.experimental import pallas as pl
from jax.experimental.pallas import tpu as pltpu
from jax.experimental.pallas import tpu_sc as plsc

DIM = 2048
UNIT_DIM = 64
N = 8192
BATCH = 16384
N_CHUNKS = DIM // UNIT_DIM
N_BLOCKS = DIM // 128

NUM_CORES = 2
NUM_SUBCORES = 16
NW = NUM_CORES * NUM_SUBCORES
TROWS = BATCH // 8
TR_PER_W = TROWS // NW
K = 8 * N_BLOCKS
NBUF = 2


def _gather_body(table_hbm, ge_hbm, go_hbm, out_hbm, ide_v, ido_v, a_v, o_v,
                 *sems):
    gse = sems[0:NBUF]
    gso = sems[NBUF:2 * NBUF]
    ssc = sems[2 * NBUF:3 * NBUF]
    wid = lax.axis_index("s") * NUM_CORES + lax.axis_index("c")
    pltpu.sync_copy(ge_hbm.at[wid], ide_v)
    pltpu.sync_copy(go_hbm.at[wid], ido_v)

    def start_gathers(j, b):
        pltpu.async_copy(table_hbm.at[ide_v.at[j]], a_v.at[b], gse[b])
        pltpu.async_copy(table_hbm.at[ido_v.at[j]], o_v.at[b], gso[b])

    def patch(b, i, _):
        for k in range(4):
            a_v[b, i, pl.ds(64 + 16 * k, 16)] = o_v[b, i, pl.ds(64 + 16 * k, 16)]
        return _

    def do_slot(r, b, start_next):
        j = r * NBUF + b
        rt = wid * TR_PER_W + j
        pltpu.make_async_copy(table_hbm.at[ide_v.at[j]], a_v.at[b],
                              gse[b]).wait()
        pltpu.make_async_copy(table_hbm.at[ido_v.at[j]], o_v.at[b],
                              gso[b]).wait()
        lax.fori_loop(0, K, lambda i, c: patch(b, i, c), 0)
        for m in range(N_BLOCKS):
            pltpu.async_copy(
                a_v.at[b, pl.ds(m * 8, 8)],
                out_hbm.at[pl.ds(rt * 8, 8), pl.ds(m * 128, 128)],
                ssc[b])
        for m in range(N_BLOCKS):
            pltpu.make_async_copy(
                a_v.at[b, pl.ds(m * 8, 8)],
                out_hbm.at[pl.ds(rt * 8, 8), pl.ds(m * 128, 128)],
                ssc[b]).wait()
        if start_next:
            start_gathers(j + NBUF, b)

    for b in range(NBUF):
        start_gathers(b, b)

    def round_body(r, carry):
        for b in range(NBUF):
            do_slot(r, b, True)
        return carry

    nrounds = TR_PER_W // NBUF
    lax.fori_loop(0, nrounds - 1, round_body, 0)
    for b in range(NBUF):
        do_slot(nrounds - 1, b, False)


@jax.jit
def _sc_gather(table, ge, go):
    mesh = plsc.VectorSubcoreMesh(core_axis_name="c", subcore_axis_name="s")
    return pl.kernel(
        _gather_body,
        out_type=jax.ShapeDtypeStruct((BATCH, DIM), jnp.float32),
        mesh=mesh,
        scratch_types=[
            pltpu.VMEM((TR_PER_W, K), jnp.int32),
            pltpu.VMEM((TR_PER_W, K), jnp.int32),
            pltpu.VMEM((NBUF, K, 128), jnp.float32),
            pltpu.VMEM((NBUF, K, 128), jnp.float32),
        ] + [pltpu.SemaphoreType.DMA] * (3 * NBUF),
        compiler_params=pltpu.CompilerParams(use_tc_tiling_on_sc=True),
    )(table, ge, go)


def kernel(inputs):
    idx_key = jax.random.key(1)
    keys = jax.vmap(lambda i: jax.random.fold_in(idx_key, i))(jnp.arange(N_CHUNKS))
    idx = jax.vmap(lambda k: jax.random.randint(k, (BATCH,), 0, N))(keys)

    v = idx * jnp.int32(N_BLOCKS) + (
        jnp.arange(N_CHUNKS, dtype=jnp.int32) // 2)[:, None]

    def arrange(vh):
        g = vh.T.reshape(TROWS, 8, N_BLOCKS).transpose(0, 2, 1)
        return g.reshape(NW, TR_PER_W, K)

    ge = arrange(v[0::2])
    go = arrange(v[1::2])

    table = inputs.reshape(BATCH * N_BLOCKS, 128)
    return _sc_gather(table, ge, go)


# speedup vs baseline: 2.6643x; 1.8069x over previous
import jax
import jax.numpy as jnp
from jax import lax
from jax.experimental import pallas as pl
from jax.experimental.pallas import tpu as pltpu
from jax.experimental.pallas import tpu_sc as plsc

DIM = 2048
UNIT_DIM = 64
N = 8192
BATCH = 16384
N_CHUNKS = DIM // UNIT_DIM
N_BLOCKS = DIM // 128

NUM_CORES = 2
NUM_SUBCORES = 16
NW = NUM_CORES * NUM_SUBCORES
S_PER_W = BATCH // NW
K = 128
ST_PER_M = S_PER_W // K
NSLOT = N_BLOCKS * ST_PER_M
NBUF = 2


def _gather_body(in_hbm, ge_hbm, go_hbm, out_hbm, ide_v, ido_v, a_v, o_v,
                 *sems):
    gse = sems[0:NBUF]
    gso = sems[NBUF:2 * NBUF]
    ssc = sems[2 * NBUF:3 * NBUF]
    wid = lax.axis_index("s") * NUM_CORES + lax.axis_index("c")
    s_base = wid * S_PER_W
    pltpu.sync_copy(ge_hbm.at[wid], ide_v)
    pltpu.sync_copy(go_hbm.at[wid], ido_v)

    def col_ref(j):
        m = j // ST_PER_M
        return in_hbm.at[:, pl.ds(m * 128, 128)]

    def start_gathers(j, b):
        pltpu.async_copy(col_ref(j).at[ide_v.at[j]], a_v.at[b], gse[b])
        pltpu.async_copy(col_ref(j).at[ido_v.at[j]], o_v.at[b], gso[b])

    def out_slice(j):
        m, st = j // ST_PER_M, j % ST_PER_M
        return out_hbm.at[pl.ds(s_base + st * K, K), pl.ds(m * 128, 128)]

    def do_slot(j, b, start_next):
        pltpu.make_async_copy(col_ref(j).at[ide_v.at[j]], a_v.at[b],
                              gse[b]).wait()
        pltpu.make_async_copy(col_ref(j).at[ido_v.at[j]], o_v.at[b],
                              gso[b]).wait()
        def patch(i, c):
            for k in range(4):
                a_v[b, i, pl.ds(64 + 16 * k, 16)] = (
                    o_v[b, i, pl.ds(64 + 16 * k, 16)])
            return c

        lax.fori_loop(0, K, patch, 0)
        pltpu.async_copy(a_v.at[b], out_slice(j), ssc[b])
        pltpu.make_async_copy(a_v.at[b], out_slice(j), ssc[b]).wait()
        if start_next:
            start_gathers(j + NBUF, b)

    for b in range(NBUF):
        start_gathers(b, b)

    def round_body(r, carry):
        for b in range(NBUF):
            do_slot(r * NBUF + b, b, True)
        return carry

    nrounds = NSLOT // NBUF
    lax.fori_loop(0, nrounds - 1, round_body, 0)
    for b in range(NBUF):
        do_slot((nrounds - 1) * NBUF + b, b, False)


@jax.jit
def _sc_gather(inputs, ge, go):
    mesh = plsc.VectorSubcoreMesh(core_axis_name="c", subcore_axis_name="s")
    return pl.kernel(
        _gather_body,
        out_type=jax.ShapeDtypeStruct((BATCH, DIM), jnp.float32),
        mesh=mesh,
        scratch_types=[
            pltpu.VMEM((NSLOT, K), jnp.int32),
            pltpu.VMEM((NSLOT, K), jnp.int32),
            pltpu.VMEM((NBUF, K, 128), jnp.float32),
            pltpu.VMEM((NBUF, K, 128), jnp.float32),
        ] + [pltpu.SemaphoreType.DMA] * (3 * NBUF),
        compiler_params=pltpu.CompilerParams(use_tc_tiling_on_sc=True),
    )(inputs, ge, go)


def kernel(inputs):
    idx_key = jax.random.key(1)
    keys = jax.vmap(lambda i: jax.random.fold_in(idx_key, i))(jnp.arange(N_CHUNKS))
    idx = jax.vmap(lambda k: jax.random.randint(k, (BATCH,), 0, N))(keys)

    def arrange(vh):
        g = vh.reshape(N_BLOCKS, NW, ST_PER_M, K).transpose(1, 0, 2, 3)
        return g.reshape(NW, NSLOT, K)

    ge = arrange(idx[0::2])
    go = arrange(idx[1::2])
    return _sc_gather(inputs, ge, go)
